# baseline (speedup 1.0000x reference)
import functools

import jax
import jax.numpy as jnp
from jax import lax
from jax.experimental import pallas as pl
from jax.experimental.pallas import tpu as pltpu
from jax.experimental.pallas import tpu_sc as plsc

NC = 2
NS = 16
NW = NC * NS
CHUNK = 128


@functools.partial(jax.jit, static_argnames=("n_rows", "d_model"))
def _embed_lookup(idx2d, weight, *, n_rows, d_model):
    n_chunks = idx2d.shape[0]
    chunks_per_w = n_chunks // NW

    mesh = plsc.VectorSubcoreMesh(
        core_axis_name="c", subcore_axis_name="s", num_cores=NC,
        num_subcores=NS)

    @functools.partial(
        pl.kernel,
        out_type=jax.ShapeDtypeStruct((n_rows, d_model), jnp.float32),
        mesh=mesh,
        scratch_types=[
            pltpu.VMEM((chunks_per_w, CHUNK), jnp.int32),
            pltpu.VMEM((CHUNK, d_model), jnp.float32),
            pltpu.SemaphoreType.DMA,
        ],
        compiler_params=pltpu.CompilerParams(use_tc_tiling_on_sc=False),
    )
    def body(idx_hbm, w_hbm, out_hbm, idx_v, rows_v, gsem):
        wid = lax.axis_index("s") * NC + lax.axis_index("c")
        chunk0 = wid * chunks_per_w
        pltpu.sync_copy(idx_hbm.at[pl.ds(chunk0, chunks_per_w)], idx_v)

        @pl.loop(0, chunks_per_w)
        def _(j):
            pltpu.async_copy(w_hbm.at[idx_v.at[j]], rows_v, gsem).wait()
            pltpu.sync_copy(rows_v, out_hbm.at[pl.ds((chunk0 + j) * CHUNK,
                                                     CHUNK)])

    return body(idx2d, weight)


def kernel(token_ids, weight):
    batch, seq = token_ids.shape
    vocab, d_model = weight.shape
    n_rows = batch * seq
    idx2d = token_ids.reshape(n_rows // CHUNK, CHUNK).astype(jnp.int32)
    out = _embed_lookup(idx2d, weight, n_rows=n_rows, d_model=d_model)
    return out.reshape(batch, seq, d_model)


# speedup vs baseline: 1.1114x; 1.1114x over previous
import functools

import jax
import jax.numpy as jnp
from jax import lax
from jax.experimental import pallas as pl
from jax.experimental.pallas import tpu as pltpu
from jax.experimental.pallas import tpu_sc as plsc

NC = 2
NS = 16
NW = NC * NS
CHUNK = 128


G = 4


@functools.partial(jax.jit, static_argnames=("n_rows", "d_model"))
def _embed_lookup(idx2d, weight, *, n_rows, d_model):
    n_chunks = idx2d.shape[0]
    chunks_per_w = n_chunks // NW
    n_groups = chunks_per_w // G
    grows = G * CHUNK

    mesh = plsc.VectorSubcoreMesh(
        core_axis_name="c", subcore_axis_name="s", num_cores=NC,
        num_subcores=NS)

    @functools.partial(
        pl.kernel,
        out_type=jax.ShapeDtypeStruct((n_rows, d_model), jnp.float32),
        mesh=mesh,
        scratch_types=[
            pltpu.VMEM((chunks_per_w, CHUNK), jnp.int32),
            pltpu.VMEM((2, grows, d_model), jnp.float32),
            pltpu.SemaphoreType.DMA,
            pltpu.SemaphoreType.DMA,
            pltpu.SemaphoreType.DMA,
            pltpu.SemaphoreType.DMA,
        ],
        compiler_params=pltpu.CompilerParams(use_tc_tiling_on_sc=False),
    )
    def body(idx_hbm, w_hbm, out_hbm, idx_v, rows_v, g0, g1, s0, s1):
        wid = lax.axis_index("s") * NC + lax.axis_index("c")
        chunk0 = wid * chunks_per_w
        row0 = chunk0 * CHUNK
        pltpu.sync_copy(idx_hbm.at[pl.ds(chunk0, chunks_per_w)], idx_v)

        gsems = (g0, g1)
        ssems = (s0, s1)

        def fire_gathers(grp, buf, sem):
            for k in range(G):
                pltpu.async_copy(w_hbm.at[idx_v.at[grp * G + k]],
                                 rows_v.at[buf, pl.ds(k * CHUNK, CHUNK)], sem)

        def drain_gathers(buf, sem):
            for k in range(G):
                pltpu.make_async_copy(
                    w_hbm.at[idx_v.at[0]],
                    rows_v.at[buf, pl.ds(k * CHUNK, CHUNK)], sem).wait()

        def fire_store(grp, buf, sem):
            pltpu.async_copy(rows_v.at[buf],
                             out_hbm.at[pl.ds(row0 + grp * grows, grows)], sem)

        def wait_store(buf, sem):
            pltpu.make_async_copy(rows_v.at[buf],
                                  out_hbm.at[pl.ds(row0, grows)], sem).wait()

        fire_gathers(0, 0, g0)

        @pl.loop(0, n_groups, step=2)
        def _(g):
            @pl.when(g > 0)
            def _():
                wait_store(1, s1)

            @pl.when(g + 1 < n_groups)
            def _():
                fire_gathers(g + 1, 1, g1)
            drain_gathers(0, g0)
            fire_store(g, 0, s0)

            wait_store(0, s0)

            @pl.when(g + 2 < n_groups)
            def _():
                fire_gathers(g + 2, 0, g0)

            @pl.when(g + 1 < n_groups)
            def _():
                drain_gathers(1, g1)
                fire_store(g + 1, 1, s1)

        if n_groups > 1:
            wait_store(1, s1)

    return body(idx2d, weight)


def kernel(token_ids, weight):
    batch, seq = token_ids.shape
    vocab, d_model = weight.shape
    n_rows = batch * seq
    idx2d = token_ids.reshape(n_rows // CHUNK, CHUNK).astype(jnp.int32)
    out = _embed_lookup(idx2d, weight, n_rows=n_rows, d_model=d_model)
    return out.reshape(batch, seq, d_model)
